# baseline (speedup 1.0000x reference)
import functools

import jax
import jax.numpy as jnp
from jax.experimental import pallas as pl
from jax.experimental.pallas import tpu as pltpu
from jax.experimental.pallas import tpu_sc as plsc

_NB = 256
_NG = 128


def _build_matrix(r, E, l, h):
    V, W = E.shape

    def body(r_ref, e_ref, l_ref, h_ref, o_ref):
        rr = r_ref[...]
        o_ref[...] = e_ref[...] + rr * l_ref[...] + (1.0 - rr) * h_ref[...]

    return pl.pallas_call(
        body,
        out_shape=jax.ShapeDtypeStruct((V, W), jnp.float32),
    )(r, E, l.reshape(1, W), h.reshape(1, W))


def kernel(idx, r, E, l, h):
    V, W = E.shape
    B, F = idx.shape
    assert B % _NB == 0 and _NB % _NG == 0
    nsteps = B // _NB

    matrix = _build_matrix(r, E, l, h)
    idx_t = idx.T.astype(jnp.int32)

    mesh = plsc.VectorSubcoreMesh(
        core_axis_name="core", subcore_axis_name="subcore"
    )

    @functools.partial(
        pl.kernel,
        out_type=jax.ShapeDtypeStruct((F, B, W), jnp.float32),
        mesh=mesh,
        scratch_types=[
            pltpu.VMEM_SHARED((V, W), jnp.float32),
            pltpu.SemaphoreType.DMA,
            pltpu.SemaphoreType.DMA,
        ],
    )
    def gather_k(x_hbm, i_hbm, o_hbm, tbl_sh, s0, s1):
        @pl.when(jax.lax.axis_index("subcore") == 0)
        def _():
            pltpu.sync_copy(x_hbm, tbl_sh)

        plsc.subcore_barrier()

        sems = (s0, s1)

        def body(i_vmem, o_vmem):
            copies = [
                pltpu.async_copy(
                    tbl_sh.at[i_vmem.at[0, pl.ds(j * _NG, _NG)]],
                    o_vmem.at[0, pl.ds(j * _NG, _NG)],
                    sems[j],
                )
                for j in range(_NB // _NG)
            ]
            for c in copies:
                c.wait()

        pltpu.emit_pipeline(
            body,
            grid=(F * nsteps,),
            in_specs=[
                pl.BlockSpec(
                    (1, _NB), index_map=lambda i: (i // nsteps, i % nsteps)
                )
            ],
            out_specs=[
                pl.BlockSpec(
                    (1, _NB, W),
                    index_map=lambda i: (i // nsteps, i % nsteps, 0),
                )
            ],
            core_axis_name=("core", "subcore"),
            dimension_semantics=(pltpu.PARALLEL,),
        )(i_hbm, o_hbm)

    out_fbw = gather_k(matrix, idx_t)
    return out_fbw.transpose(1, 0, 2)
